# baseline (speedup 1.0000x reference)
import functools

import jax
import jax.numpy as jnp
from jax import lax
from jax.experimental import pallas as pl
from jax.experimental.pallas import tpu as pltpu
from jax.experimental.pallas import tpu_sc as plsc

VOCAB = 1000000
B = 4096
L = 200
D = 64
O = 2

NC, NS = 2, 16
NW = NC * NS
RPW = B // NW
HALF = 100
CPR = L // HALF

TCHUNK = 32768
TSEMI = TCHUNK // 2
TGRID = (VOCAB + TCHUNK - 1) // TCHUNK
VROWS = TGRID * TCHUNK


def _transpose_table_tc(tableT, e1, e2):
    def body(t_ref, e1_ref, e2_ref, o_ref):
        dn = (((0,), (0,)), ((), ()))
        xa = t_ref[:, : TSEMI]
        xb = t_ref[:, TSEMI:]
        o_ref[...] = (
            lax.dot_general(xa, e1_ref[...], dn,
                            preferred_element_type=jnp.float32)
            + lax.dot_general(xb, e2_ref[...], dn,
                              preferred_element_type=jnp.float32)
        )

    return pl.pallas_call(
        body,
        grid=(TGRID,),
        in_specs=[
            pl.BlockSpec((D, TCHUNK), lambda i: (0, i)),
            pl.BlockSpec((D, 2 * D), lambda i: (0, 0)),
            pl.BlockSpec((D, 2 * D), lambda i: (0, 0)),
        ],
        out_specs=pl.BlockSpec((TSEMI, 2 * D), lambda i: (i, 0)),
        out_shape=jax.ShapeDtypeStruct((VROWS // 2, 2 * D), jnp.float32),
    )(tableT, e1, e2)


def _pooled_sum_sc(idx3, table):
    mesh = plsc.VectorSubcoreMesh(core_axis_name="c", subcore_axis_name="s")

    @functools.partial(
        pl.kernel,
        mesh=mesh,
        compiler_params=pltpu.CompilerParams(use_tc_tiling_on_sc=False),
        out_type=jax.ShapeDtypeStruct((B, D), jnp.float32),
        scratch_types=[
            pltpu.VMEM((RPW * CPR, HALF), jnp.int32),
            pltpu.VMEM((2, CPR, HALF, D), jnp.float32),
            pltpu.VMEM((RPW, D), jnp.float32),
            pltpu.SemaphoreType.DMA,
            pltpu.SemaphoreType.DMA,
        ],
    )
    def k(idx_hbm, table_hbm, out_hbm, idx_v, buf, out_v, sem0, sem1):
        wid = lax.axis_index("s") * NC + lax.axis_index("c")
        sems = (sem0, sem1)

        pltpu.sync_copy(idx_hbm.at[wid], idx_v)

        def issue(r, slot):
            for h in range(CPR):
                pltpu.make_async_copy(
                    table_hbm.at[idx_v.at[CPR * r + h]],
                    buf.at[slot, h],
                    sems[slot],
                ).start()

        def wait(slot):
            for h in range(CPR):
                pltpu.make_async_copy(
                    table_hbm.at[idx_v.at[0]],
                    buf.at[slot, h],
                    sems[slot],
                ).wait()

        def accum(r, slot):
            zero = jnp.zeros((16,), jnp.float32)

            def body(g, acc):
                new = []
                for j in range(D // 16):
                    v = acc[j]
                    for u in range(4):
                        for h in range(CPR):
                            v = v + buf[slot, h, 4 * g + u, pl.ds(16 * j, 16)]
                    new.append(v)
                return tuple(new)

            acc = lax.fori_loop(0, HALF // 4, body, (zero,) * (D // 16))
            for j in range(D // 16):
                out_v[r, pl.ds(16 * j, 16)] = acc[j]

        issue(0, 0)

        def outer(g):
            for s in range(2):
                r = g + s

                @pl.when(r + 1 < RPW)
                def _():
                    issue(r + 1, 1 - s)

                wait(s)
                accum(r, s)

        pl.loop(0, RPW, step=2)(outer)

        pltpu.sync_copy(out_v, out_hbm.at[pl.ds(wid * RPW, RPW)])

    return k(idx3, table)


def _head_tc(pooled, W, b2):
    def body(p_ref, w_ref, b_ref, o_ref):
        mean = p_ref[...] * (1.0 / L)
        z = lax.dot_general(
            mean, w_ref[...], (((1,), (1,)), ((), ())),
            preferred_element_type=jnp.float32,
        ) + b_ref[...]
        s = jax.nn.sigmoid(z)
        m = jnp.max(s, axis=-1, keepdims=True)
        lse = m + jnp.log(jnp.sum(jnp.exp(s - m), axis=-1, keepdims=True))
        o_ref[...] = s - lse

    return pl.pallas_call(
        body,
        out_shape=jax.ShapeDtypeStruct((B, O), jnp.float32),
    )(pooled, W, b2)


def kernel(batch, lengths, emb_table, W, b):
    del lengths
    v = batch.astype(jnp.int32)
    vr = (v // TCHUNK) * TCHUNK + (v % TSEMI) * 2 + (v // TSEMI) % 2
    idx3 = vr.reshape(NW, RPW * CPR, HALF)
    eye = jnp.eye(D, dtype=jnp.float32)
    zero = jnp.zeros((D, D), jnp.float32)
    e1 = jnp.concatenate([eye, zero], axis=1)
    e2 = jnp.concatenate([zero, eye], axis=1)
    table_rm = _transpose_table_tc(emb_table.T, e1, e2).reshape(VROWS, D)
    pooled = _pooled_sum_sc(idx3, table_rm)
    return _head_tc(pooled, W, b.reshape(1, O))


# speedup vs baseline: 1.0097x; 1.0097x over previous
import functools

import jax
import jax.numpy as jnp
from jax import lax
from jax.experimental import pallas as pl
from jax.experimental.pallas import tpu as pltpu
from jax.experimental.pallas import tpu_sc as plsc

VOCAB = 1000000
B = 4096
L = 200
D = 64
O = 2

NC, NS = 2, 16
NW = NC * NS
RPW = B // NW
HALF = 100
CPR = L // HALF

TCHUNK = 40960
TSEMI = TCHUNK // 2
TGRID = (VOCAB + TCHUNK - 1) // TCHUNK
VROWS = TGRID * TCHUNK


def _transpose_table_tc(tableT, e1, e2):
    def body(t_ref, e1_ref, e2_ref, o_ref):
        dn = (((0,), (0,)), ((), ()))
        xa = t_ref[:, : TSEMI]
        xb = t_ref[:, TSEMI:]
        o_ref[...] = (
            lax.dot_general(xa, e1_ref[...], dn,
                            preferred_element_type=jnp.float32)
            + lax.dot_general(xb, e2_ref[...], dn,
                              preferred_element_type=jnp.float32)
        )

    return pl.pallas_call(
        body,
        grid=(TGRID,),
        in_specs=[
            pl.BlockSpec((D, TCHUNK), lambda i: (0, i)),
            pl.BlockSpec((D, 2 * D), lambda i: (0, 0)),
            pl.BlockSpec((D, 2 * D), lambda i: (0, 0)),
        ],
        out_specs=pl.BlockSpec((TSEMI, 2 * D), lambda i: (i, 0)),
        out_shape=jax.ShapeDtypeStruct((VROWS // 2, 2 * D), jnp.float32),
    )(tableT, e1, e2)


def _pooled_sum_sc(idx3, table):
    mesh = plsc.VectorSubcoreMesh(core_axis_name="c", subcore_axis_name="s")

    @functools.partial(
        pl.kernel,
        mesh=mesh,
        compiler_params=pltpu.CompilerParams(use_tc_tiling_on_sc=False),
        out_type=jax.ShapeDtypeStruct((B, D), jnp.float32),
        scratch_types=[
            pltpu.VMEM((RPW * CPR, HALF), jnp.int32),
            pltpu.VMEM((2, CPR, HALF, D), jnp.float32),
            pltpu.VMEM((RPW, D), jnp.float32),
            pltpu.SemaphoreType.DMA,
            pltpu.SemaphoreType.DMA,
        ],
    )
    def k(idx_hbm, table_hbm, out_hbm, idx_v, buf, out_v, sem0, sem1):
        wid = lax.axis_index("s") * NC + lax.axis_index("c")
        sems = (sem0, sem1)

        pltpu.sync_copy(idx_hbm.at[wid], idx_v)

        def issue(r, slot):
            for h in range(CPR):
                pltpu.make_async_copy(
                    table_hbm.at[idx_v.at[CPR * r + h]],
                    buf.at[slot, h],
                    sems[slot],
                ).start()

        def wait(slot):
            for h in range(CPR):
                pltpu.make_async_copy(
                    table_hbm.at[idx_v.at[0]],
                    buf.at[slot, h],
                    sems[slot],
                ).wait()

        def accum(r, slot):
            zero = jnp.zeros((16,), jnp.float32)

            def body(g, acc):
                new = []
                for j in range(D // 16):
                    v = acc[j]
                    for u in range(4):
                        for h in range(CPR):
                            v = v + buf[slot, h, 4 * g + u, pl.ds(16 * j, 16)]
                    new.append(v)
                return tuple(new)

            acc = lax.fori_loop(0, HALF // 4, body, (zero,) * (D // 16))
            for j in range(D // 16):
                out_v[r, pl.ds(16 * j, 16)] = acc[j]

        issue(0, 0)

        def outer(g):
            for s in range(2):
                r = g + s

                @pl.when(r + 1 < RPW)
                def _():
                    issue(r + 1, 1 - s)

                wait(s)
                accum(r, s)

        pl.loop(0, RPW, step=2)(outer)

        pltpu.sync_copy(out_v, out_hbm.at[pl.ds(wid * RPW, RPW)])

    return k(idx3, table)


def _head_tc(pooled, W, b2):
    def body(p_ref, w_ref, b_ref, o_ref):
        mean = p_ref[...] * (1.0 / L)
        z = lax.dot_general(
            mean, w_ref[...], (((1,), (1,)), ((), ())),
            preferred_element_type=jnp.float32,
        ) + b_ref[...]
        s = jax.nn.sigmoid(z)
        m = jnp.max(s, axis=-1, keepdims=True)
        lse = m + jnp.log(jnp.sum(jnp.exp(s - m), axis=-1, keepdims=True))
        o_ref[...] = s - lse

    return pl.pallas_call(
        body,
        out_shape=jax.ShapeDtypeStruct((B, O), jnp.float32),
    )(pooled, W, b2)


def kernel(batch, lengths, emb_table, W, b):
    del lengths
    v = batch.astype(jnp.int32)
    vr = (v // TCHUNK) * TCHUNK + (v % TSEMI) * 2 + (v // TSEMI) % 2
    idx3 = vr.reshape(NW, RPW * CPR, HALF)
    eye = jnp.eye(D, dtype=jnp.float32)
    zero = jnp.zeros((D, D), jnp.float32)
    e1 = jnp.concatenate([eye, zero], axis=1)
    e2 = jnp.concatenate([zero, eye], axis=1)
    table_rm = _transpose_table_tc(emb_table.T, e1, e2).reshape(VROWS, D)
    pooled = _pooled_sum_sc(idx3, table_rm)
    return _head_tc(pooled, W, b.reshape(1, O))
